# baseline (speedup 1.0000x reference)
import functools

import jax
import jax.numpy as jnp
from jax import lax
from jax.experimental import pallas as pl
from jax.experimental.pallas import tpu as pltpu
from jax.experimental.pallas import tpu_sc as plsc

N_NODES = 10000
D_IN = 128
H_OUT = 64
N_EDGES = 320000

NC = 2
NS = 16
CHUNK = 256
NK = 80
E_PAD = NK * NS * CHUNK
SLICE = 640
ACC_ROWS = SLICE * NS
DUMMY_DST = N_NODES
CW = 16

ROW_BLK = 1000
GRID = N_NODES // ROW_BLK


def _make_seg_sum(first_layer):
  out_type = [jax.ShapeDtypeStruct((ACC_ROWS, H_OUT), jnp.float32),
              jax.ShapeDtypeStruct((ACC_ROWS, H_OUT), jnp.float32)]
  if first_layer:
    out_type += [jax.ShapeDtypeStruct((ACC_ROWS, CW), jnp.float32),
                 jax.ShapeDtypeStruct((ACC_ROWS, CW), jnp.float32)]

  scratch = dict(
      acc_s=pltpu.VMEM_SHARED((ACC_ROWS, H_OUT), jnp.float32),
      rows_v=[pltpu.VMEM((CHUNK, H_OUT), jnp.float32) for _ in range(4)],
      idx_sv=[pltpu.VMEM((CHUNK,), jnp.int32) for _ in range(4)],
      idx_dv=[pltpu.VMEM((CHUNK,), jnp.int32) for _ in range(4)],
      cnt_v=pltpu.VMEM((256, CW), jnp.float32),
      sem_g=[pltpu.SemaphoreType.DMA for _ in range(4)],
      sem_s=[pltpu.SemaphoreType.DMA for _ in range(4)],
      sem_is=[pltpu.SemaphoreType.DMA for _ in range(4)],
      sem_id=[pltpu.SemaphoreType.DMA for _ in range(4)],
  )
  if first_layer:
    scratch.update(
        cnt_s=pltpu.VMEM_SHARED((ACC_ROWS, CW), jnp.float32),
        ones_v=pltpu.VMEM((CHUNK, CW), jnp.float32),
        sem_c=[pltpu.SemaphoreType.DMA for _ in range(4)],
    )

  mesh = plsc.VectorSubcoreMesh(core_axis_name="c", subcore_axis_name="s")

  @functools.partial(
      pl.kernel, out_type=out_type, mesh=mesh, scratch_types=scratch,
      compiler_params=pltpu.CompilerParams(use_tc_tiling_on_sc=False))
  def seg_sum(y_a, y_b, src_a, dst_a, src_b, dst_b, zero64, zero16, ones16,
              self_a, self_b, cnt_a_in, cnt_b_in,
              *args, acc_s, rows_v, idx_sv, idx_dv, cnt_v, sem_g, sem_s,
              sem_is, sem_id, cnt_s=None, ones_v=None, sem_c=None):
    if first_layer:
      out_a, out_b, cnt_a, cnt_b = args
    else:
      out_a, out_b = args
    c = lax.axis_index("c")
    s = lax.axis_index("s")
    my_rows = pl.ds(s * SLICE, SLICE)

    pltpu.sync_copy(zero64, acc_s.at[my_rows])
    if first_layer:
      pltpu.sync_copy(zero16, cnt_s.at[my_rows])
      pltpu.sync_copy(ones16, ones_v)
    plsc.subcore_barrier()

    for core_id, (y, src_t, dst_t) in enumerate(
        ((y_a, src_a, dst_a), (y_b, src_b, dst_b))):
      @pl.when(c == core_id)
      def _():
        base = s * NK

        def load_idx_s(m, p):
          pltpu.async_copy(src_t.at[base + m], idx_sv[p], sem_is[p])

        def wait_idx_s(m, p):
          pltpu.make_async_copy(src_t.at[base + m], idx_sv[p],
                                sem_is[p]).wait()

        def load_idx_d(m, p):
          pltpu.async_copy(dst_t.at[base + m], idx_dv[p], sem_id[p])

        def wait_idx_d(m, p):
          pltpu.make_async_copy(dst_t.at[base + m], idx_dv[p],
                                sem_id[p]).wait()

        def fire_gather(p):
          pltpu.async_copy(y.at[idx_sv[p]], rows_v[p], sem_g[p])

        def wait_gather(p):
          pltpu.make_async_copy(y.at[idx_sv[p]], rows_v[p], sem_g[p]).wait()

        def fire_scatter(p):
          pltpu.async_copy(rows_v[p], acc_s.at[idx_dv[p]], sem_s[p], add=True)
          if first_layer:
            pltpu.async_copy(ones_v, cnt_s.at[idx_dv[p]], sem_c[p], add=True)

        def drain_scatter(p):
          pltpu.make_async_copy(rows_v[p], acc_s.at[idx_dv[p]],
                                sem_s[p]).wait()
          if first_layer:
            pltpu.make_async_copy(ones_v, cnt_s.at[idx_dv[p]],
                                  sem_c[p]).wait()

        for m in range(4):
          load_idx_s(m, m)
        load_idx_d(0, 0)
        load_idx_d(1, 1)
        wait_idx_s(0, 0)
        fire_gather(0)
        wait_idx_s(1, 1)
        fire_gather(1)

        nk4 = NK // 4

        def body(k4, carry):
          for b in range(4):
            k = k4 * 4 + b
            p = b
            j2 = (b + 2) % 4

            def head():
              drain_scatter(j2)

            if b < 2:
              @pl.when(k4 > 0)
              def _():
                head()
            else:
              head()

            def stage_ahead():
              load_idx_d(k + 2, j2)
              wait_idx_s(k + 2, j2)
              fire_gather(j2)

            if b < 2:
              stage_ahead()
            else:
              @pl.when(k4 < nk4 - 1)
              def _():
                stage_ahead()

            wait_gather(p)
            @pl.when(k4 < nk4 - 1)
            def _():
              load_idx_s(k + 4, p)
            wait_idx_d(k, p)
            fire_scatter(p)
          return carry

        lax.fori_loop(0, nk4, body, 0)
        drain_scatter((NK - 2) % 4)
        drain_scatter((NK - 1) % 4)

    plsc.subcore_barrier()

    epi = ((out_a, self_a, cnt_a_in, cnt_a if first_layer else None),
           (out_b, self_b, cnt_b_in, cnt_b if first_layer else None))
    for core_id, (out_t, self_t, cnt_in, cnt_out) in enumerate(epi):
      @pl.when(c == core_id)
      def _():
        if first_layer:
          pltpu.sync_copy(cnt_s.at[my_rows], cnt_out.at[my_rows])
        for blk in range(3):
          n = 256 if blk < 2 else SLICE - 512
          r0 = s * SLICE + blk * 256
          cnt_src = cnt_s if first_layer else cnt_in
          pltpu.sync_copy(cnt_src.at[pl.ds(r0, n)], cnt_v.at[pl.ds(0, n)])
          pltpu.sync_copy(acc_s.at[pl.ds(r0, n)], rows_v[0].at[pl.ds(0, n)])
          pltpu.sync_copy(self_t.at[pl.ds(r0, n)], rows_v[1].at[pl.ds(0, n)])

          def rbody(r, carry):
            cnt_row = cnt_v[r, pl.ds(0, 16)]
            rec = 1.0 / jnp.maximum(jnp.full((16,), cnt_row[0]), 1.0)
            for j in range(H_OUT // 16):
              sl = pl.ds(j * 16, 16)
              v = rows_v[0][r, sl] * rec + rows_v[1][r, sl]
              if first_layer:
                v = jnp.maximum(v, 0.0)
              rows_v[0][r, sl] = v
            return carry

          lax.fori_loop(0, n, rbody, 0)
          pltpu.sync_copy(rows_v[0].at[pl.ds(0, n)], out_t.at[pl.ds(r0, n)])

  return seg_sum


_seg_sum_l1 = _make_seg_sum(True)
_seg_sum_l2 = _make_seg_sum(False)


def _mm(x, w):
  return jnp.dot(x, w, preferred_element_type=jnp.float32)


def _tc1_body(xu, xi, w1l_a, w1r_a, b1_a, w1l_b, w1r_b, b1_b,
              y1u, y1i, s1i, s1u):
  xu_b = xu[...]
  xi_b = xi[...]
  y1u[...] = _mm(xu_b, w1l_a[...])
  y1i[...] = _mm(xi_b, w1l_b[...])
  s1i[...] = _mm(xi_b, w1r_a[...]) + b1_a[...]
  s1u[...] = _mm(xu_b, w1r_b[...]) + b1_b[...]


def _tc2_body(h_i, h_u, w2l_a, w2r_a, b2_a, w2l_b, w2r_b, b2_b,
              z2u, z2i, s2i, s2u):
  hi_b = h_i[...]
  hu_b = h_u[...]
  z2u[...] = _mm(hu_b, w2l_a[...])
  z2i[...] = _mm(hi_b, w2l_b[...])
  s2i[...] = _mm(hi_b, w2r_a[...]) + b2_a[...]
  s2u[...] = _mm(hu_b, w2r_b[...]) + b2_b[...]


def _row_spec(width):
  return pl.BlockSpec((ROW_BLK, width), lambda i: (i, 0))


def _full_spec(shape):
  return pl.BlockSpec(shape, lambda i: tuple(0 for _ in shape))


def _pad_edges(ei):
  ei = ei.astype(jnp.int32)
  pad = E_PAD - N_EDGES
  src = jnp.concatenate([ei[0], jnp.zeros((pad,), jnp.int32)])
  dst = jnp.concatenate([ei[1], jnp.full((pad,), DUMMY_DST, jnp.int32)])
  return (src.reshape(E_PAD // CHUNK, CHUNK),
          dst.reshape(E_PAD // CHUNK, CHUNK))


def kernel(x_user, x_item, edge_index_u2i, edge_index_i2u,
           W1l_u2i, W1r_u2i, b1_u2i, W1l_i2u, W1r_i2u, b1_i2u,
           W2l_u2i, W2r_u2i, b2_u2i, W2l_i2u, W2r_i2u, b2_i2u):
  src_a, dst_a = _pad_edges(edge_index_u2i)
  src_b, dst_b = _pad_edges(edge_index_i2u)
  zero64 = jnp.zeros((SLICE, H_OUT), jnp.float32)
  zero16 = jnp.zeros((SLICE, CW), jnp.float32)
  ones16 = jnp.ones((CHUNK, CW), jnp.float32)

  f32 = jnp.float32
  blk = jax.ShapeDtypeStruct((ACC_ROWS, H_OUT), f32)
  cnt_dummy = jnp.zeros((ACC_ROWS, CW), f32)

  y1u, y1i, s1i, s1u = pl.pallas_call(
      _tc1_body,
      grid=(GRID,),
      in_specs=[_row_spec(D_IN), _row_spec(D_IN),
                _full_spec((D_IN, H_OUT)), _full_spec((D_IN, H_OUT)),
                _full_spec((1, H_OUT)),
                _full_spec((D_IN, H_OUT)), _full_spec((D_IN, H_OUT)),
                _full_spec((1, H_OUT))],
      out_specs=[_row_spec(H_OUT)] * 4,
      out_shape=[blk] * 4,
  )(x_user, x_item, W1l_u2i, W1r_u2i, b1_u2i.reshape(1, H_OUT),
    W1l_i2u, W1r_i2u, b1_i2u.reshape(1, H_OUT))

  h_i, h_u, cnt_i, cnt_u = _seg_sum_l1(
      y1u, y1i, src_a, dst_a, src_b, dst_b, zero64, zero16, ones16,
      s1i, s1u, cnt_dummy, cnt_dummy)

  z2u, z2i, s2i, s2u = pl.pallas_call(
      _tc2_body,
      grid=(GRID,),
      in_specs=[_row_spec(H_OUT), _row_spec(H_OUT),
                _full_spec((H_OUT, H_OUT)), _full_spec((H_OUT, H_OUT)),
                _full_spec((1, H_OUT)),
                _full_spec((H_OUT, H_OUT)), _full_spec((H_OUT, H_OUT)),
                _full_spec((1, H_OUT))],
      out_specs=[_row_spec(H_OUT)] * 4,
      out_shape=[blk] * 4,
  )(h_i, h_u, W2l_u2i, W2r_u2i, b2_u2i.reshape(1, H_OUT),
    W2l_i2u, W2r_i2u, b2_i2u.reshape(1, H_OUT))

  o_i, o_u = _seg_sum_l2(
      z2u, z2i, src_a, dst_a, src_b, dst_b, zero64, zero16, ones16,
      s2i, s2u, cnt_i, cnt_u)

  return (o_u[:N_NODES], o_i[:N_NODES])


# speedup vs baseline: 1.0244x; 1.0244x over previous
import functools

import jax
import jax.numpy as jnp
from jax import lax
from jax.experimental import pallas as pl
from jax.experimental.pallas import tpu as pltpu
from jax.experimental.pallas import tpu_sc as plsc

N_NODES = 10000
D_IN = 128
H_OUT = 64
N_EDGES = 320000

NC = 2
NS = 16
CHUNK = 256
NK = 80
E_PAD = NK * NS * CHUNK
SLICE = 640
ACC_ROWS = SLICE * NS
DUMMY_DST = N_NODES
CW = 8

ROW_BLK = 1000
GRID = N_NODES // ROW_BLK


def _make_seg_sum(first_layer):
  out_type = [jax.ShapeDtypeStruct((ACC_ROWS, H_OUT), jnp.float32),
              jax.ShapeDtypeStruct((ACC_ROWS, H_OUT), jnp.float32)]
  if first_layer:
    out_type += [jax.ShapeDtypeStruct((ACC_ROWS, CW), jnp.float32),
                 jax.ShapeDtypeStruct((ACC_ROWS, CW), jnp.float32)]

  scratch = dict(
      acc_s=pltpu.VMEM_SHARED((ACC_ROWS, H_OUT), jnp.float32),
      rows_v=[pltpu.VMEM((CHUNK, H_OUT), jnp.float32) for _ in range(4)],
      idx_sv=[pltpu.VMEM((CHUNK,), jnp.int32) for _ in range(4)],
      idx_dv=[pltpu.VMEM((CHUNK,), jnp.int32) for _ in range(4)],
      sem_g=[pltpu.SemaphoreType.DMA for _ in range(4)],
      sem_s=[pltpu.SemaphoreType.DMA for _ in range(4)],
      sem_is=[pltpu.SemaphoreType.DMA for _ in range(4)],
      sem_id=[pltpu.SemaphoreType.DMA for _ in range(4)],
  )
  if first_layer:
    scratch.update(
        cnt_s=pltpu.VMEM_SHARED((ACC_ROWS, CW), jnp.float32),
        ones_v=pltpu.VMEM((CHUNK, CW), jnp.float32),
        sem_c=[pltpu.SemaphoreType.DMA for _ in range(4)],
    )

  mesh = plsc.VectorSubcoreMesh(core_axis_name="c", subcore_axis_name="s")

  @functools.partial(
      pl.kernel, out_type=out_type, mesh=mesh, scratch_types=scratch,
      compiler_params=pltpu.CompilerParams(use_tc_tiling_on_sc=False))
  def seg_sum(y_a, y_b, src_a, dst_a, src_b, dst_b, zero64, zero16, ones16,
              *args, acc_s, rows_v, idx_sv, idx_dv, sem_g, sem_s,
              sem_is, sem_id, cnt_s=None, ones_v=None, sem_c=None):
    if first_layer:
      sum_a, sum_b, cnt_a, cnt_b = args
    else:
      sum_a, sum_b = args
    c = lax.axis_index("c")
    s = lax.axis_index("s")
    my_rows = pl.ds(s * SLICE, SLICE)

    pltpu.sync_copy(zero64, acc_s.at[my_rows])
    if first_layer:
      pltpu.sync_copy(zero16, cnt_s.at[my_rows])
      pltpu.sync_copy(ones16, ones_v)
    plsc.subcore_barrier()

    for core_id, (y, src_t, dst_t) in enumerate(
        ((y_a, src_a, dst_a), (y_b, src_b, dst_b))):
      @pl.when(c == core_id)
      def _():
        base = s * NK

        def load_idx_s(m, p):
          pltpu.async_copy(src_t.at[base + m], idx_sv[p], sem_is[p])

        def wait_idx_s(m, p):
          pltpu.make_async_copy(src_t.at[base + m], idx_sv[p],
                                sem_is[p]).wait()

        def load_idx_d(m, p):
          pltpu.async_copy(dst_t.at[base + m], idx_dv[p], sem_id[p])

        def wait_idx_d(m, p):
          pltpu.make_async_copy(dst_t.at[base + m], idx_dv[p],
                                sem_id[p]).wait()

        def fire_gather(p):
          pltpu.async_copy(y.at[idx_sv[p]], rows_v[p], sem_g[p])

        def wait_gather(p):
          pltpu.make_async_copy(y.at[idx_sv[p]], rows_v[p], sem_g[p]).wait()

        def fire_scatter(p):
          pltpu.async_copy(rows_v[p], acc_s.at[idx_dv[p]], sem_s[p], add=True)
          if first_layer:
            pltpu.async_copy(ones_v, cnt_s.at[idx_dv[p]], sem_c[p], add=True)

        def drain_scatter(p):
          pltpu.make_async_copy(rows_v[p], acc_s.at[idx_dv[p]],
                                sem_s[p]).wait()
          if first_layer:
            pltpu.make_async_copy(ones_v, cnt_s.at[idx_dv[p]],
                                  sem_c[p]).wait()

        for m in range(4):
          load_idx_s(m, m)
        load_idx_d(0, 0)
        load_idx_d(1, 1)
        wait_idx_s(0, 0)
        fire_gather(0)
        wait_idx_s(1, 1)
        fire_gather(1)

        nk4 = NK // 4

        def body(k4, carry):
          for b in range(4):
            k = k4 * 4 + b
            p = b
            j2 = (b + 2) % 4

            def head():
              drain_scatter(j2)

            if b < 2:
              @pl.when(k4 > 0)
              def _():
                head()
            else:
              head()

            def stage_ahead():
              load_idx_d(k + 2, j2)
              wait_idx_s(k + 2, j2)
              fire_gather(j2)

            if b < 2:
              stage_ahead()
            else:
              @pl.when(k4 < nk4 - 1)
              def _():
                stage_ahead()

            wait_gather(p)
            @pl.when(k4 < nk4 - 1)
            def _():
              load_idx_s(k + 4, p)
            wait_idx_d(k, p)
            fire_scatter(p)
          return carry

        lax.fori_loop(0, nk4, body, 0)
        drain_scatter((NK - 2) % 4)
        drain_scatter((NK - 1) % 4)

    plsc.subcore_barrier()

    @pl.when(c == 0)
    def _():
      pltpu.sync_copy(acc_s.at[my_rows], sum_a.at[my_rows])
      if first_layer:
        pltpu.sync_copy(cnt_s.at[my_rows], cnt_a.at[my_rows])

    @pl.when(c == 1)
    def _():
      pltpu.sync_copy(acc_s.at[my_rows], sum_b.at[my_rows])
      if first_layer:
        pltpu.sync_copy(cnt_s.at[my_rows], cnt_b.at[my_rows])

  return seg_sum


_seg_sum_l1 = _make_seg_sum(True)
_seg_sum_l2 = _make_seg_sum(False)


def _mm(x, w):
  return jnp.dot(x, w, preferred_element_type=jnp.float32)


def _tc1_body(xu, xi, w1l_a, w1r_a, b1_a, w1l_b, w1r_b, b1_b,
              y1u, y1i, s1i, s1u):
  xu_b = xu[...]
  xi_b = xi[...]
  y1u[...] = _mm(xu_b, w1l_a[...])
  y1i[...] = _mm(xi_b, w1l_b[...])
  s1i[...] = _mm(xi_b, w1r_a[...]) + b1_a[...]
  s1u[...] = _mm(xu_b, w1r_b[...]) + b1_b[...]


def _tc2_body(sum_i, cnt_i, s1i, sum_u, cnt_u, s1u,
              w2l_a, w2r_a, b2_a, w2l_b, w2r_b, b2_b,
              z2u, z2i, s2i, s2u):
  ci = jnp.maximum(cnt_i[:, 0:1], 1.0)
  cu = jnp.maximum(cnt_u[:, 0:1], 1.0)
  h_i = jnp.maximum(sum_i[...] / ci + s1i[...], 0.0)
  h_u = jnp.maximum(sum_u[...] / cu + s1u[...], 0.0)
  z2u[...] = _mm(h_u, w2l_a[...])
  z2i[...] = _mm(h_i, w2l_b[...])
  s2i[...] = _mm(h_i, w2r_a[...]) + b2_a[...]
  s2u[...] = _mm(h_u, w2r_b[...]) + b2_b[...]


def _tc3_body(sum_i, cnt_i, s2i, sum_u, cnt_u, s2u, o_user, o_item):
  ci = jnp.maximum(cnt_i[:, 0:1], 1.0)
  cu = jnp.maximum(cnt_u[:, 0:1], 1.0)
  o_item[...] = sum_i[...] / ci + s2i[...]
  o_user[...] = sum_u[...] / cu + s2u[...]


def _row_spec(width):
  return pl.BlockSpec((ROW_BLK, width), lambda i: (i, 0))


def _full_spec(shape):
  return pl.BlockSpec(shape, lambda i: tuple(0 for _ in shape))


def _pad_edges(ei):
  ei = ei.astype(jnp.int32)
  pad = E_PAD - N_EDGES
  src = jnp.concatenate([ei[0], jnp.zeros((pad,), jnp.int32)])
  dst = jnp.concatenate([ei[1], jnp.full((pad,), DUMMY_DST, jnp.int32)])
  return (src.reshape(E_PAD // CHUNK, CHUNK),
          dst.reshape(E_PAD // CHUNK, CHUNK))


def kernel(x_user, x_item, edge_index_u2i, edge_index_i2u,
           W1l_u2i, W1r_u2i, b1_u2i, W1l_i2u, W1r_i2u, b1_i2u,
           W2l_u2i, W2r_u2i, b2_u2i, W2l_i2u, W2r_i2u, b2_i2u):
  src_a, dst_a = _pad_edges(edge_index_u2i)
  src_b, dst_b = _pad_edges(edge_index_i2u)
  zero64 = jnp.zeros((SLICE, H_OUT), jnp.float32)
  zero16 = jnp.zeros((SLICE, CW), jnp.float32)
  ones16 = jnp.ones((CHUNK, CW), jnp.float32)

  f32 = jnp.float32
  blk = jax.ShapeDtypeStruct((N_NODES, H_OUT), f32)

  y1u, y1i, s1i, s1u = pl.pallas_call(
      _tc1_body,
      grid=(GRID,),
      in_specs=[_row_spec(D_IN), _row_spec(D_IN),
                _full_spec((D_IN, H_OUT)), _full_spec((D_IN, H_OUT)),
                _full_spec((1, H_OUT)),
                _full_spec((D_IN, H_OUT)), _full_spec((D_IN, H_OUT)),
                _full_spec((1, H_OUT))],
      out_specs=[_row_spec(H_OUT)] * 4,
      out_shape=[blk] * 4,
  )(x_user, x_item, W1l_u2i, W1r_u2i, b1_u2i.reshape(1, H_OUT),
    W1l_i2u, W1r_i2u, b1_i2u.reshape(1, H_OUT))

  sum1_i, sum1_u, cnt_i, cnt_u = _seg_sum_l1(
      y1u, y1i, src_a, dst_a, src_b, dst_b, zero64, zero16, ones16)

  z2u, z2i, s2i, s2u = pl.pallas_call(
      _tc2_body,
      grid=(GRID,),
      in_specs=[_row_spec(H_OUT), _row_spec(CW), _row_spec(H_OUT),
                _row_spec(H_OUT), _row_spec(CW), _row_spec(H_OUT),
                _full_spec((H_OUT, H_OUT)), _full_spec((H_OUT, H_OUT)),
                _full_spec((1, H_OUT)),
                _full_spec((H_OUT, H_OUT)), _full_spec((H_OUT, H_OUT)),
                _full_spec((1, H_OUT))],
      out_specs=[_row_spec(H_OUT)] * 4,
      out_shape=[blk] * 4,
  )(sum1_i, cnt_i, s1i, sum1_u, cnt_u, s1u,
    W2l_u2i, W2r_u2i, b2_u2i.reshape(1, H_OUT),
    W2l_i2u, W2r_i2u, b2_i2u.reshape(1, H_OUT))

  sum2_i, sum2_u = _seg_sum_l2(
      z2u, z2i, src_a, dst_a, src_b, dst_b, zero64, zero16, ones16)

  o_user, o_item = pl.pallas_call(
      _tc3_body,
      grid=(GRID,),
      in_specs=[_row_spec(H_OUT), _row_spec(CW), _row_spec(H_OUT),
                _row_spec(H_OUT), _row_spec(CW), _row_spec(H_OUT)],
      out_specs=[_row_spec(H_OUT)] * 2,
      out_shape=[blk] * 2,
  )(sum2_i, cnt_i, s2i, sum2_u, cnt_u, s2u)

  return (o_user, o_item)


# speedup vs baseline: 1.0475x; 1.0225x over previous
import functools

import jax
import jax.numpy as jnp
from jax import lax
from jax.experimental import pallas as pl
from jax.experimental.pallas import tpu as pltpu
from jax.experimental.pallas import tpu_sc as plsc

N_NODES = 10000
D_IN = 128
H_OUT = 64
N_EDGES = 320000

NC = 2
NS = 16
CHUNK = 512
NK = 40
E_PAD = NK * NS * CHUNK
SLICE = 640
ACC_ROWS = SLICE * NS
DUMMY_DST = N_NODES
CW = 8

ROW_BLK = 1000
GRID = N_NODES // ROW_BLK


def _make_seg_sum(first_layer):
  out_type = [jax.ShapeDtypeStruct((ACC_ROWS, H_OUT), jnp.float32),
              jax.ShapeDtypeStruct((ACC_ROWS, H_OUT), jnp.float32)]
  if first_layer:
    out_type += [jax.ShapeDtypeStruct((ACC_ROWS, CW), jnp.float32),
                 jax.ShapeDtypeStruct((ACC_ROWS, CW), jnp.float32)]

  scratch = dict(
      acc_s=pltpu.VMEM_SHARED((ACC_ROWS, H_OUT), jnp.float32),
      rows_v=[pltpu.VMEM((CHUNK, H_OUT), jnp.float32) for _ in range(2)],
      idx_sv=[pltpu.VMEM((CHUNK,), jnp.int32) for _ in range(2)],
      idx_dv=[pltpu.VMEM((CHUNK,), jnp.int32) for _ in range(2)],
      sem_g=[pltpu.SemaphoreType.DMA for _ in range(2)],
      sem_s=[pltpu.SemaphoreType.DMA for _ in range(2)],
      sem_is=[pltpu.SemaphoreType.DMA for _ in range(2)],
      sem_id=[pltpu.SemaphoreType.DMA for _ in range(2)],
  )
  if first_layer:
    scratch.update(
        cnt_s=pltpu.VMEM_SHARED((ACC_ROWS, CW), jnp.float32),
        ones_v=pltpu.VMEM((CHUNK, CW), jnp.float32),
        sem_c=[pltpu.SemaphoreType.DMA for _ in range(2)],
    )

  mesh = plsc.VectorSubcoreMesh(core_axis_name="c", subcore_axis_name="s")

  @functools.partial(
      pl.kernel, out_type=out_type, mesh=mesh, scratch_types=scratch,
      compiler_params=pltpu.CompilerParams(use_tc_tiling_on_sc=False))
  def seg_sum(y_a, y_b, src_a, dst_a, src_b, dst_b, zero64, zero16, ones16,
              *args, acc_s, rows_v, idx_sv, idx_dv, sem_g, sem_s,
              sem_is, sem_id, cnt_s=None, ones_v=None, sem_c=None):
    if first_layer:
      sum_a, sum_b, cnt_a, cnt_b = args
    else:
      sum_a, sum_b = args
    c = lax.axis_index("c")
    s = lax.axis_index("s")
    my_rows = pl.ds(s * SLICE, SLICE)

    pltpu.sync_copy(zero64, acc_s.at[my_rows])
    if first_layer:
      pltpu.sync_copy(zero16, cnt_s.at[my_rows])
      pltpu.sync_copy(ones16, ones_v)
    plsc.subcore_barrier()

    for core_id, (y, src_t, dst_t) in enumerate(
        ((y_a, src_a, dst_a), (y_b, src_b, dst_b))):
      @pl.when(c == core_id)
      def _():
        base = s * NK

        def load_idx_s(m, p):
          pltpu.async_copy(src_t.at[base + m], idx_sv[p], sem_is[p])

        def wait_idx_s(m, p):
          pltpu.make_async_copy(src_t.at[base + m], idx_sv[p],
                                sem_is[p]).wait()

        def load_idx_d(m, p):
          pltpu.async_copy(dst_t.at[base + m], idx_dv[p], sem_id[p])

        def wait_idx_d(m, p):
          pltpu.make_async_copy(dst_t.at[base + m], idx_dv[p],
                                sem_id[p]).wait()

        def fire_gather(p):
          pltpu.async_copy(y.at[idx_sv[p]], rows_v[p], sem_g[p])

        def wait_gather(p):
          pltpu.make_async_copy(y.at[idx_sv[p]], rows_v[p], sem_g[p]).wait()

        def fire_scatter(p):
          pltpu.async_copy(rows_v[p], acc_s.at[idx_dv[p]], sem_s[p], add=True)
          if first_layer:
            pltpu.async_copy(ones_v, cnt_s.at[idx_dv[p]], sem_c[p], add=True)

        def drain_scatter(p):
          pltpu.make_async_copy(rows_v[p], acc_s.at[idx_dv[p]],
                                sem_s[p]).wait()
          if first_layer:
            pltpu.make_async_copy(ones_v, cnt_s.at[idx_dv[p]],
                                  sem_c[p]).wait()

        load_idx_s(0, 0)
        load_idx_d(0, 0)
        load_idx_s(1, 1)
        load_idx_d(1, 1)
        wait_idx_s(0, 0)
        fire_gather(0)

        nkh = NK // 2

        def body(k2, carry):
          for b in range(2):
            k = k2 * 2 + b
            p, q = b, 1 - b

            def steady_head():
              drain_scatter(q)
              load_idx_d(k + 1, q)

            if b == 0:
              @pl.when(k2 > 0)
              def _():
                steady_head()
            else:
              drain_scatter(q)
              @pl.when(k2 < nkh - 1)
              def _():
                load_idx_d(k + 1, q)

            def start_next_gather():
              wait_idx_s(k + 1, q)
              fire_gather(q)

            if b == 0:
              start_next_gather()
            else:
              @pl.when(k2 < nkh - 1)
              def _():
                start_next_gather()

            wait_gather(p)
            @pl.when(k2 < nkh - 1)
            def _():
              load_idx_s(k + 2, p)
            wait_idx_d(k, p)
            fire_scatter(p)
          return carry

        lax.fori_loop(0, nkh, body, 0)
        drain_scatter((NK - 1) % 2)

    plsc.subcore_barrier()

    @pl.when(c == 0)
    def _():
      pltpu.sync_copy(acc_s.at[my_rows], sum_a.at[my_rows])
      if first_layer:
        pltpu.sync_copy(cnt_s.at[my_rows], cnt_a.at[my_rows])

    @pl.when(c == 1)
    def _():
      pltpu.sync_copy(acc_s.at[my_rows], sum_b.at[my_rows])
      if first_layer:
        pltpu.sync_copy(cnt_s.at[my_rows], cnt_b.at[my_rows])

  return seg_sum


_seg_sum_l1 = _make_seg_sum(True)
_seg_sum_l2 = _make_seg_sum(False)


def _mm(x, w):
  return jnp.dot(x, w, preferred_element_type=jnp.float32)


def _tc1_body(xu, xi, w1l_a, w1r_a, b1_a, w1l_b, w1r_b, b1_b,
              y1u, y1i, s1i, s1u):
  xu_b = xu[...]
  xi_b = xi[...]
  y1u[...] = _mm(xu_b, w1l_a[...])
  y1i[...] = _mm(xi_b, w1l_b[...])
  s1i[...] = _mm(xi_b, w1r_a[...]) + b1_a[...]
  s1u[...] = _mm(xu_b, w1r_b[...]) + b1_b[...]


def _tc2_body(sum_i, cnt_i, s1i, sum_u, cnt_u, s1u,
              w2l_a, w2r_a, b2_a, w2l_b, w2r_b, b2_b,
              z2u, z2i, s2i, s2u):
  ci = jnp.maximum(cnt_i[:, 0:1], 1.0)
  cu = jnp.maximum(cnt_u[:, 0:1], 1.0)
  h_i = jnp.maximum(sum_i[...] / ci + s1i[...], 0.0)
  h_u = jnp.maximum(sum_u[...] / cu + s1u[...], 0.0)
  z2u[...] = _mm(h_u, w2l_a[...])
  z2i[...] = _mm(h_i, w2l_b[...])
  s2i[...] = _mm(h_i, w2r_a[...]) + b2_a[...]
  s2u[...] = _mm(h_u, w2r_b[...]) + b2_b[...]


def _tc3_body(sum_i, cnt_i, s2i, sum_u, cnt_u, s2u, o_user, o_item):
  ci = jnp.maximum(cnt_i[:, 0:1], 1.0)
  cu = jnp.maximum(cnt_u[:, 0:1], 1.0)
  o_item[...] = sum_i[...] / ci + s2i[...]
  o_user[...] = sum_u[...] / cu + s2u[...]


def _row_spec(width):
  return pl.BlockSpec((ROW_BLK, width), lambda i: (i, 0))


def _full_spec(shape):
  return pl.BlockSpec(shape, lambda i: tuple(0 for _ in shape))


def _pad_edges(ei):
  ei = ei.astype(jnp.int32)
  pad = E_PAD - N_EDGES
  src = jnp.concatenate([ei[0], jnp.zeros((pad,), jnp.int32)])
  dst = jnp.concatenate([ei[1], jnp.full((pad,), DUMMY_DST, jnp.int32)])
  return (src.reshape(E_PAD // CHUNK, CHUNK),
          dst.reshape(E_PAD // CHUNK, CHUNK))


def kernel(x_user, x_item, edge_index_u2i, edge_index_i2u,
           W1l_u2i, W1r_u2i, b1_u2i, W1l_i2u, W1r_i2u, b1_i2u,
           W2l_u2i, W2r_u2i, b2_u2i, W2l_i2u, W2r_i2u, b2_i2u):
  src_a, dst_a = _pad_edges(edge_index_u2i)
  src_b, dst_b = _pad_edges(edge_index_i2u)
  zero64 = jnp.zeros((SLICE, H_OUT), jnp.float32)
  zero16 = jnp.zeros((SLICE, CW), jnp.float32)
  ones16 = jnp.ones((CHUNK, CW), jnp.float32)

  f32 = jnp.float32
  blk = jax.ShapeDtypeStruct((N_NODES, H_OUT), f32)

  y1u, y1i, s1i, s1u = pl.pallas_call(
      _tc1_body,
      grid=(GRID,),
      in_specs=[_row_spec(D_IN), _row_spec(D_IN),
                _full_spec((D_IN, H_OUT)), _full_spec((D_IN, H_OUT)),
                _full_spec((1, H_OUT)),
                _full_spec((D_IN, H_OUT)), _full_spec((D_IN, H_OUT)),
                _full_spec((1, H_OUT))],
      out_specs=[_row_spec(H_OUT)] * 4,
      out_shape=[blk] * 4,
  )(x_user, x_item, W1l_u2i, W1r_u2i, b1_u2i.reshape(1, H_OUT),
    W1l_i2u, W1r_i2u, b1_i2u.reshape(1, H_OUT))

  sum1_i, sum1_u, cnt_i, cnt_u = _seg_sum_l1(
      y1u, y1i, src_a, dst_a, src_b, dst_b, zero64, zero16, ones16)

  z2u, z2i, s2i, s2u = pl.pallas_call(
      _tc2_body,
      grid=(GRID,),
      in_specs=[_row_spec(H_OUT), _row_spec(CW), _row_spec(H_OUT),
                _row_spec(H_OUT), _row_spec(CW), _row_spec(H_OUT),
                _full_spec((H_OUT, H_OUT)), _full_spec((H_OUT, H_OUT)),
                _full_spec((1, H_OUT)),
                _full_spec((H_OUT, H_OUT)), _full_spec((H_OUT, H_OUT)),
                _full_spec((1, H_OUT))],
      out_specs=[_row_spec(H_OUT)] * 4,
      out_shape=[blk] * 4,
  )(sum1_i, cnt_i, s1i, sum1_u, cnt_u, s1u,
    W2l_u2i, W2r_u2i, b2_u2i.reshape(1, H_OUT),
    W2l_i2u, W2r_i2u, b2_i2u.reshape(1, H_OUT))

  sum2_i, sum2_u = _seg_sum_l2(
      z2u, z2i, src_a, dst_a, src_b, dst_b, zero64, zero16, ones16)

  o_user, o_item = pl.pallas_call(
      _tc3_body,
      grid=(GRID,),
      in_specs=[_row_spec(H_OUT), _row_spec(CW), _row_spec(H_OUT),
                _row_spec(H_OUT), _row_spec(CW), _row_spec(H_OUT)],
      out_specs=[_row_spec(H_OUT)] * 2,
      out_shape=[blk] * 2,
  )(sum2_i, cnt_i, s2i, sum2_u, cnt_u, s2u)

  return (o_user, o_item)
